# baseline (speedup 1.0000x reference)
import functools

import jax
import jax.numpy as jnp
from jax.experimental import pallas as pl
from jax.experimental.pallas import tpu as pltpu
from jax.experimental.pallas import tpu_sc as plsc

_INTERPRET = False

K = 16
EPS = 1e-5


def _knn_body(nsample, n_db, offset_scale, refs):
    q_ref, db_ref, idx_ref, dsel_ref = refs
    b = pl.program_id(0)
    q = q_ref[0]
    db = db_ref[0]
    qn = jnp.sum(q * q, axis=0)[:, None]
    dbn = jnp.sum(db * db, axis=0)[None, :]
    qd = jax.lax.dot_general(q, db, (((0,), (0,)), ((), ())),
                             preferred_element_type=jnp.float32)
    d = qn + dbn - 2.0 * qd
    iota = jax.lax.broadcasted_iota(jnp.int32, d.shape, 1)
    cols = []
    dcols = []
    for _ in range(nsample):
        m = jnp.min(d, axis=1, keepdims=True)
        ik = jnp.argmin(d, axis=1)
        cols.append(ik[:, None])
        dcols.append(jnp.maximum(m, 0.0))
        d = jnp.where(iota == ik[:, None], jnp.float32(jnp.inf), d)
    idx = jnp.concatenate(cols, axis=1)
    idx_ref[0] = idx + b * offset_scale
    euc = jnp.sqrt(jnp.concatenate(dcols, axis=1) + 1e-20)
    dsel_ref[0] = euc


def _knn(query_xyz, db_xyz, offset_scale):
    B, _, S = query_xyz.shape
    N = db_xyz.shape[2]
    TS = min(256, S)
    body = functools.partial(_knn_body, K, N, offset_scale)
    return pl.pallas_call(
        lambda *refs: body(refs),
        grid=(B, S // TS),
        in_specs=[
            pl.BlockSpec((1, 3, TS), lambda b, t: (b, 0, t)),
            pl.BlockSpec((1, 3, N), lambda b, t: (b, 0, 0)),
        ],
        out_specs=[pl.BlockSpec((1, TS, K), lambda b, t: (b, t, 0)),
                   pl.BlockSpec((1, TS, K), lambda b, t: (b, t, 0))],
        out_shape=[jax.ShapeDtypeStruct((B, S, K), jnp.int32),
                   jax.ShapeDtypeStruct((B, S, K), jnp.float32)],
        interpret=_INTERPRET,
    )(query_xyz, db_xyz)


def _gather(table, idx_flat):
    M = idx_flat.shape[0]
    D = table.shape[1]
    NC = 2
    NW = 32
    per_w = M // NW
    CH = 128
    mesh = plsc.VectorSubcoreMesh(core_axis_name="c", subcore_axis_name="s")

    @functools.partial(
        pl.kernel, mesh=mesh,
        out_type=jax.ShapeDtypeStruct((M, D), jnp.float32),
        scratch_types=[
            pltpu.VMEM((CH,), jnp.int32),
            pltpu.VMEM((CH, D), jnp.float32),
            pltpu.SemaphoreType.DMA,
        ],
    )
    def k(table_hbm, idx_hbm, out_hbm, idx_v, rows_v, sem):
        wid = jax.lax.axis_index("s") * NC + jax.lax.axis_index("c")
        base = wid * per_w

        def body(i, carry):
            off = base + i * CH
            pltpu.sync_copy(idx_hbm.at[pl.ds(off, CH)], idx_v)
            pltpu.async_copy(table_hbm.at[idx_v], rows_v, sem).wait()
            pltpu.sync_copy(rows_v, out_hbm.at[pl.ds(off, CH)])
            return carry

        jax.lax.fori_loop(0, per_w // CH, body, 0)

    return k(table, idx_flat)


def _act(z, ab_ref):
    a = ab_ref[0:1, :]
    b = ab_ref[1:2, :]
    return jnp.maximum(z * a + b, 0.0)


def _accum_stats(z, out_ref):
    sz = jnp.sum(z, axis=0)
    sq = jnp.sum(z * z, axis=0)
    st = jnp.concatenate([sz[None, :], sq[None, :]], axis=0)
    first = (pl.program_id(0) == 0) & (pl.program_id(1) == 0)

    @pl.when(first)
    def _():
        out_ref[...] = st

    @pl.when(jnp.logical_not(first))
    def _():
        out_ref[...] = out_ref[...] + st


def _dot(x, w):
    return jax.lax.dot_general(x, w, (((1,), (0,)), ((), ())),
                               preferred_element_type=jnp.float32)


def _qadd(zr, zq, TQ):
    C = zr.shape[1]
    return (zr.reshape(TQ, K, C) + zq[:, None, :]).reshape(TQ * K, C)


def _segsum(x, TQ):
    C = x.shape[1]
    return jnp.sum(x.reshape(TQ, K, C), axis=1)


def _h1_body(phase, TQ, refs):
    (wcat_ref, g_ref, d_ref, WA, WB, CE, W2, W3, W4a, W4b, W5,
     ab1, abx, ab2, ab3, ab4, ab5, out_ref) = refs
    wcat = wcat_ref[0]
    g = g_ref[0]
    euc = d_ref[0]
    t = _qadd(_dot(g, WB[...]) + euc * CE[...], _dot(wcat, WA[...]), TQ)
    z1 = t[:, 0:128]
    if phase == 0:
        _accum_stats(t, out_ref)
        return
    y1 = _act(z1, ab1)
    z2 = _dot(y1, W2[...])
    if phase == 1:
        _accum_stats(z2, out_ref)
        return
    y2 = _act(z2, ab2)
    z3 = _dot(y2, W3[...])
    if phase == 2:
        _accum_stats(z3, out_ref)
        return
    y3 = _act(z3, ab3)
    e = _act(t[:, 128:192], abx)
    z4 = _dot(e, W4a[...]) + _dot(y3, W4b[...])
    if phase == 3:
        _accum_stats(z4, out_ref)
        return
    y4 = _act(z4, ab4)
    z5 = _dot(y4, W5[...])
    if phase == 4:
        _accum_stats(z5, out_ref)
        return
    y5 = _act(z5, ab5)
    w = jnp.exp(y5)
    denom = _segsum(w, TQ)
    num = _segsum(w * y3, TQ)
    outq = num / denom
    pad = jnp.zeros((TQ, 61), jnp.float32)
    out_ref[0] = jnp.concatenate([outq, wcat[:, 0:3], pad], axis=1)


def _h2_body(phase, TQ, refs):
    (wcat_ref, g_ref, d_ref, WA2, WB2, CE2, W6a, W6b, W6c, W7,
     abx, ab6, ab7, out_ref) = refs
    wcat = wcat_ref[0]
    g = g_ref[0]
    euc = d_ref[0]
    ex = _qadd(_dot(g, WB2[...]) + euc * CE2[...], _dot(wcat, WA2[...]), TQ)
    if phase == 0:
        _accum_stats(ex, out_ref)
        return
    e = _act(ex, abx)
    z6 = _qadd(_dot(e, W6a[...]) + _dot(g, W6c[...]),
               _dot(wcat, W6b[...]), TQ)
    if phase == 1:
        _accum_stats(z6, out_ref)
        return
    y6 = _act(z6, ab6)
    z7 = _dot(y6, W7[...])
    if phase == 2:
        _accum_stats(z7, out_ref)
        return
    y7 = _act(z7, ab7)
    w = jnp.exp(y7)
    gf = g[:, 0:64]
    denom = _segsum(w, TQ)
    num = _segsum(w * gf, TQ)
    out_ref[0] = num / denom


def _full_spec(shape):
    nd = len(shape)
    return pl.BlockSpec(shape, lambda b, t, _n=nd: (0,) * _n)


def _run_half(body_fn, phase, wcat, g3, d3, weights, abs_, out_shape, out_spec, TQ):
    B, S, _ = wcat.shape
    in_specs = [
        pl.BlockSpec((1, TQ, 67), lambda b, t: (b, t, 0)),
        pl.BlockSpec((1, TQ * K, 128), lambda b, t: (b, t, 0)),
        pl.BlockSpec((1, TQ * K, 1), lambda b, t: (b, t, 0)),
    ]
    in_specs += [_full_spec(w.shape) for w in weights]
    in_specs += [_full_spec(a.shape) for a in abs_]
    return pl.pallas_call(
        lambda *refs: body_fn(phase, TQ, refs),
        grid=(B, S // TQ),
        in_specs=in_specs,
        out_specs=out_spec,
        out_shape=out_shape,
        interpret=_INTERPRET,
    )(wcat, g3, d3, *weights, *abs_)


def _stats_out(C):
    return (jax.ShapeDtypeStruct((2, C), jnp.float32),
            pl.BlockSpec((2, C), lambda b, t: (0, 0)))


def _make_ab(stats, gamma, beta, count):
    s, q = stats[0], stats[1]
    mean = s / count
    var = q / count - mean * mean
    a = gamma / jnp.sqrt(var + EPS)
    b = beta - mean * a
    return jnp.stack([a, b])


def kernel(warped_xyz, warped_points, f2_xyz, f2_points,
           mlp1_params, xyz1_params, xyz2_params, mlp2_params, mlp3_params):
    B, _, S = warped_xyz.shape
    N = f2_xyz.shape[2]
    f32 = jnp.float32
    TQ = min(128, S)
    count = float(B * S * K)

    wxyz_t = jnp.transpose(warped_xyz, (0, 2, 1))
    wcat = jnp.concatenate([wxyz_t, jnp.transpose(warped_points, (0, 2, 1))],
                           axis=2)
    table1 = jnp.concatenate(
        [jnp.transpose(f2_points, (0, 2, 1)),
         jnp.transpose(f2_xyz, (0, 2, 1)),
         jnp.zeros((B, N, 61), f32)], axis=2).reshape(B * N, 128)

    (W1, g1_, b1_), (W2, g2_, b2_), (W3, g3_, b3_) = mlp1_params
    ((Wx1, gx1, bx1),) = xyz1_params
    ((Wx2, gx2, bx2),) = xyz2_params
    (W4, g4_, b4_), (W5, g5_, b5_) = mlp2_params
    (W6, g6_, b6_), (W7, g7_, b7_) = mlp3_params
    W1t = W1.T
    Wx1t = Wx1.T
    WA = jnp.concatenate([
        jnp.concatenate([W1t[0:3] - W1t[6:9], W1t[10:74]], axis=0),
        jnp.concatenate([Wx1t[0:3] - Wx1t[6:9], jnp.zeros((64, 64), f32)],
                        axis=0)], axis=1)
    WB = jnp.concatenate([
        jnp.concatenate([W1t[74:138], W1t[3:6] + W1t[6:9],
                         jnp.zeros((61, 128), f32)], axis=0),
        jnp.concatenate([jnp.zeros((64, 64), f32), Wx1t[3:6] + Wx1t[6:9],
                         jnp.zeros((61, 64), f32)], axis=0)], axis=1)
    CE = jnp.concatenate([W1t[9:10], Wx1t[9:10]], axis=1)
    W2t, W3t = W2.T, W3.T
    W4t = W4.T
    W4a, W4b = W4t[0:64], W4t[64:128]
    W5t = W5.T
    Wx2t = Wx2.T
    WA2 = jnp.concatenate([Wx2t[0:3] - Wx2t[6:9],
                           jnp.zeros((64, 64), f32)], axis=0)
    WB2 = jnp.concatenate([jnp.zeros((64, 64), f32), Wx2t[3:6] + Wx2t[6:9],
                           jnp.zeros((61, 64), f32)], axis=0)
    CE2 = Wx2t[9:10]
    W6t = W6.T
    W6a = W6t[0:64]
    W6b = jnp.concatenate([jnp.zeros((3, 128), f32), W6t[64:128]], axis=0)
    W6c = jnp.concatenate([W6t[128:192], jnp.zeros((64, 128), f32)], axis=0)
    W7t = W7.T

    h1_w = [WA, WB, CE, W2t, W3t, W4a, W4b, W5t]
    h2_w = [WA2, WB2, CE2, W6a, W6b, W6c, W7t]

    z128 = jnp.zeros((2, 128), f32)
    z64 = jnp.zeros((2, 64), f32)

    idx1, dsel1 = _knn(warped_xyz, f2_xyz, N)
    g1 = _gather(table1, idx1.reshape(-1)).reshape(B, S * K, 128)
    d1 = dsel1.reshape(B, S * K, 1)

    ab = [z128, z64, z64, z64, z128, z64]
    sh1, sp1 = _stats_out(128)
    shx, spx = _stats_out(64)
    sht, spt = _stats_out(192)
    st = _run_half(_h1_body, 0, wcat, g1, d1, h1_w, ab, sht, spt, TQ)
    ab[0] = _make_ab(st[:, 0:128], g1_, b1_, count)
    ab[1] = _make_ab(st[:, 128:192], gx1, bx1, count)
    st = _run_half(_h1_body, 1, wcat, g1, d1, h1_w, ab, shx, spx, TQ)
    ab[2] = _make_ab(st, g2_, b2_, count)
    st = _run_half(_h1_body, 2, wcat, g1, d1, h1_w, ab, shx, spx, TQ)
    ab[3] = _make_ab(st, g3_, b3_, count)
    st = _run_half(_h1_body, 3, wcat, g1, d1, h1_w, ab, sh1, sp1, TQ)
    ab[4] = _make_ab(st, g4_, b4_, count)
    st = _run_half(_h1_body, 4, wcat, g1, d1, h1_w, ab, shx, spx, TQ)
    ab[5] = _make_ab(st, g5_, b5_, count)
    pf = _run_half(_h1_body, 5, wcat, g1, d1, h1_w, ab,
                   jax.ShapeDtypeStruct((B, S, 128), f32),
                   pl.BlockSpec((1, TQ, 128), lambda b, t: (b, t, 0)), TQ)

    idx2, dsel2 = _knn(warped_xyz, warped_xyz, S)
    g2 = _gather(pf.reshape(B * S, 128), idx2.reshape(-1)).reshape(B, S * K, 128)
    d2 = dsel2.reshape(B, S * K, 1)

    ab2_ = [z64, z128, z64]
    st = _run_half(_h2_body, 0, wcat, g2, d2, h2_w, ab2_, shx, spx, TQ)
    ab2_[0] = _make_ab(st, gx2, bx2, count)
    st = _run_half(_h2_body, 1, wcat, g2, d2, h2_w, ab2_, sh1, sp1, TQ)
    ab2_[1] = _make_ab(st, g6_, b6_, count)
    st = _run_half(_h2_body, 2, wcat, g2, d2, h2_w, ab2_, shx, spx, TQ)
    ab2_[2] = _make_ab(st, g7_, b7_, count)
    out = _run_half(_h2_body, 3, wcat, g2, d2, h2_w, ab2_,
                    jax.ShapeDtypeStruct((B, S, 64), f32),
                    pl.BlockSpec((1, TQ, 64), lambda b, t: (b, t, 0)), TQ)

    return jnp.transpose(out, (0, 2, 1))


# speedup vs baseline: 1.0715x; 1.0715x over previous
import functools

import jax
import jax.numpy as jnp
from jax.experimental import pallas as pl
from jax.experimental.pallas import tpu as pltpu
from jax.experimental.pallas import tpu_sc as plsc

_INTERPRET = False

K = 16
EPS = 1e-5


def _knn_body(nsample, n_db, offset_scale, refs):
    q_ref, db_ref, idx_ref, dsel_ref = refs
    b = pl.program_id(0)
    q = q_ref[0]
    db = db_ref[0]
    qn = jnp.sum(q * q, axis=0)[:, None]
    dbn = jnp.sum(db * db, axis=0)[None, :]
    qd = jax.lax.dot_general(q, db, (((0,), (0,)), ((), ())),
                             preferred_element_type=jnp.float32)
    d = qn + dbn - 2.0 * qd
    iota = jax.lax.broadcasted_iota(jnp.int32, d.shape, 1)
    cols = []
    dcols = []
    for _ in range(nsample):
        m = jnp.min(d, axis=1, keepdims=True)
        cand = jnp.where(d <= m, iota, n_db)
        ik = jnp.min(cand, axis=1)
        cols.append(ik[:, None])
        dcols.append(jnp.maximum(m, 0.0))
        d = jnp.where(iota == ik[:, None], jnp.float32(jnp.inf), d)
    idx = jnp.concatenate(cols, axis=1)
    idx_ref[0] = idx + b * offset_scale
    euc = jnp.sqrt(jnp.concatenate(dcols, axis=1) + 1e-20)
    dsel_ref[0] = euc


def _knn(query_xyz, db_xyz, offset_scale):
    B, _, S = query_xyz.shape
    N = db_xyz.shape[2]
    TS = min(256, S)
    body = functools.partial(_knn_body, K, N, offset_scale)
    return pl.pallas_call(
        lambda *refs: body(refs),
        grid=(B, S // TS),
        in_specs=[
            pl.BlockSpec((1, 3, TS), lambda b, t: (b, 0, t)),
            pl.BlockSpec((1, 3, N), lambda b, t: (b, 0, 0)),
        ],
        out_specs=[pl.BlockSpec((1, TS, K), lambda b, t: (b, t, 0)),
                   pl.BlockSpec((1, TS, K), lambda b, t: (b, t, 0))],
        out_shape=[jax.ShapeDtypeStruct((B, S, K), jnp.int32),
                   jax.ShapeDtypeStruct((B, S, K), jnp.float32)],
        interpret=_INTERPRET,
    )(query_xyz, db_xyz)


def _gather(table, idx_flat):
    M = idx_flat.shape[0]
    D = table.shape[1]
    NC = 2
    NW = 32
    per_w = M // NW
    CH = 128
    mesh = plsc.VectorSubcoreMesh(core_axis_name="c", subcore_axis_name="s")

    @functools.partial(
        pl.kernel, mesh=mesh,
        out_type=jax.ShapeDtypeStruct((M, D), jnp.float32),
        scratch_types=[
            pltpu.VMEM((CH,), jnp.int32),
            pltpu.VMEM((CH, D), jnp.float32),
            pltpu.SemaphoreType.DMA,
        ],
    )
    def k(table_hbm, idx_hbm, out_hbm, idx_v, rows_v, sem):
        wid = jax.lax.axis_index("s") * NC + jax.lax.axis_index("c")
        base = wid * per_w

        def body(i, carry):
            off = base + i * CH
            pltpu.sync_copy(idx_hbm.at[pl.ds(off, CH)], idx_v)
            pltpu.async_copy(table_hbm.at[idx_v], rows_v, sem).wait()
            pltpu.sync_copy(rows_v, out_hbm.at[pl.ds(off, CH)])
            return carry

        jax.lax.fori_loop(0, per_w // CH, body, 0)

    return k(table, idx_flat)


def _act(z, ab_ref):
    a = ab_ref[0:1, :]
    b = ab_ref[1:2, :]
    return jnp.maximum(z * a + b, 0.0)


def _accum_stats(z, out_ref):
    sz = jnp.sum(z, axis=0)
    sq = jnp.sum(z * z, axis=0)
    st = jnp.concatenate([sz[None, :], sq[None, :]], axis=0)
    first = (pl.program_id(0) == 0) & (pl.program_id(1) == 0)

    @pl.when(first)
    def _():
        out_ref[...] = st

    @pl.when(jnp.logical_not(first))
    def _():
        out_ref[...] = out_ref[...] + st


def _dot(x, w):
    return jax.lax.dot_general(x, w, (((1,), (0,)), ((), ())),
                               preferred_element_type=jnp.float32)


def _qadd(zr, zq, TQ):
    C = zr.shape[1]
    return (zr.reshape(TQ, K, C) + zq[:, None, :]).reshape(TQ * K, C)


def _segsum(x, TQ):
    C = x.shape[1]
    return jnp.sum(x.reshape(TQ, K, C), axis=1)


def _h1_body(phase, TQ, refs):
    (wcat_ref, g_ref, d_ref, WA, WB, CE, W2, W3, W4a, W4b, W5,
     ab1, abx, ab2, ab3, ab4, ab5, out_ref) = refs
    wcat = wcat_ref[0]
    g = g_ref[0]
    euc = d_ref[0]
    t = _qadd(_dot(g, WB[...]) + euc * CE[...], _dot(wcat, WA[...]), TQ)
    z1 = t[:, 0:128]
    if phase == 0:
        _accum_stats(t, out_ref)
        return
    y1 = _act(z1, ab1)
    z2 = _dot(y1, W2[...])
    if phase == 1:
        _accum_stats(z2, out_ref)
        return
    y2 = _act(z2, ab2)
    z3 = _dot(y2, W3[...])
    if phase == 2:
        _accum_stats(z3, out_ref)
        return
    y3 = _act(z3, ab3)
    e = _act(t[:, 128:192], abx)
    z4 = _dot(e, W4a[...]) + _dot(y3, W4b[...])
    if phase == 3:
        _accum_stats(z4, out_ref)
        return
    y4 = _act(z4, ab4)
    z5 = _dot(y4, W5[...])
    if phase == 4:
        _accum_stats(z5, out_ref)
        return
    y5 = _act(z5, ab5)
    w = jnp.exp(y5)
    denom = _segsum(w, TQ)
    num = _segsum(w * y3, TQ)
    outq = num / denom
    pad = jnp.zeros((TQ, 61), jnp.float32)
    out_ref[0] = jnp.concatenate([outq, wcat[:, 0:3], pad], axis=1)


def _h2_body(phase, TQ, refs):
    (wcat_ref, g_ref, d_ref, WA2, WB2, CE2, W6a, W6b, W6c, W7,
     abx, ab6, ab7, out_ref) = refs
    wcat = wcat_ref[0]
    g = g_ref[0]
    euc = d_ref[0]
    ex = _qadd(_dot(g, WB2[...]) + euc * CE2[...], _dot(wcat, WA2[...]), TQ)
    if phase == 0:
        _accum_stats(ex, out_ref)
        return
    e = _act(ex, abx)
    z6 = _qadd(_dot(e, W6a[...]) + _dot(g, W6c[...]),
               _dot(wcat, W6b[...]), TQ)
    if phase == 1:
        _accum_stats(z6, out_ref)
        return
    y6 = _act(z6, ab6)
    z7 = _dot(y6, W7[...])
    if phase == 2:
        _accum_stats(z7, out_ref)
        return
    y7 = _act(z7, ab7)
    w = jnp.exp(y7)
    gf = g[:, 0:64]
    denom = _segsum(w, TQ)
    num = _segsum(w * gf, TQ)
    out_ref[0] = num / denom


def _full_spec(shape):
    nd = len(shape)
    return pl.BlockSpec(shape, lambda b, t, _n=nd: (0,) * _n)


def _run_half(body_fn, phase, wcat, g3, d3, weights, abs_, out_shape, out_spec, TQ):
    B, S, _ = wcat.shape
    in_specs = [
        pl.BlockSpec((1, TQ, 67), lambda b, t: (b, t, 0)),
        pl.BlockSpec((1, TQ * K, 128), lambda b, t: (b, t, 0)),
        pl.BlockSpec((1, TQ * K, 1), lambda b, t: (b, t, 0)),
    ]
    in_specs += [_full_spec(w.shape) for w in weights]
    in_specs += [_full_spec(a.shape) for a in abs_]
    return pl.pallas_call(
        lambda *refs: body_fn(phase, TQ, refs),
        grid=(B, S // TQ),
        in_specs=in_specs,
        out_specs=out_spec,
        out_shape=out_shape,
        interpret=_INTERPRET,
    )(wcat, g3, d3, *weights, *abs_)


def _stats_out(C):
    return (jax.ShapeDtypeStruct((2, C), jnp.float32),
            pl.BlockSpec((2, C), lambda b, t: (0, 0)))


def _make_ab(stats, gamma, beta, count):
    s, q = stats[0], stats[1]
    mean = s / count
    var = q / count - mean * mean
    a = gamma / jnp.sqrt(var + EPS)
    b = beta - mean * a
    return jnp.stack([a, b])


def kernel(warped_xyz, warped_points, f2_xyz, f2_points,
           mlp1_params, xyz1_params, xyz2_params, mlp2_params, mlp3_params):
    B, _, S = warped_xyz.shape
    N = f2_xyz.shape[2]
    f32 = jnp.float32
    TQ = min(128, S)
    count = float(B * S * K)

    wxyz_t = jnp.transpose(warped_xyz, (0, 2, 1))
    wcat = jnp.concatenate([wxyz_t, jnp.transpose(warped_points, (0, 2, 1))],
                           axis=2)
    table1 = jnp.concatenate(
        [jnp.transpose(f2_points, (0, 2, 1)),
         jnp.transpose(f2_xyz, (0, 2, 1)),
         jnp.zeros((B, N, 61), f32)], axis=2).reshape(B * N, 128)

    (W1, g1_, b1_), (W2, g2_, b2_), (W3, g3_, b3_) = mlp1_params
    ((Wx1, gx1, bx1),) = xyz1_params
    ((Wx2, gx2, bx2),) = xyz2_params
    (W4, g4_, b4_), (W5, g5_, b5_) = mlp2_params
    (W6, g6_, b6_), (W7, g7_, b7_) = mlp3_params
    W1t = W1.T
    Wx1t = Wx1.T
    WA = jnp.concatenate([
        jnp.concatenate([W1t[0:3] - W1t[6:9], W1t[10:74]], axis=0),
        jnp.concatenate([Wx1t[0:3] - Wx1t[6:9], jnp.zeros((64, 64), f32)],
                        axis=0)], axis=1)
    WB = jnp.concatenate([
        jnp.concatenate([W1t[74:138], W1t[3:6] + W1t[6:9],
                         jnp.zeros((61, 128), f32)], axis=0),
        jnp.concatenate([jnp.zeros((64, 64), f32), Wx1t[3:6] + Wx1t[6:9],
                         jnp.zeros((61, 64), f32)], axis=0)], axis=1)
    CE = jnp.concatenate([W1t[9:10], Wx1t[9:10]], axis=1)
    W2t, W3t = W2.T, W3.T
    W4t = W4.T
    W4a, W4b = W4t[0:64], W4t[64:128]
    W5t = W5.T
    Wx2t = Wx2.T
    WA2 = jnp.concatenate([Wx2t[0:3] - Wx2t[6:9],
                           jnp.zeros((64, 64), f32)], axis=0)
    WB2 = jnp.concatenate([jnp.zeros((64, 64), f32), Wx2t[3:6] + Wx2t[6:9],
                           jnp.zeros((61, 64), f32)], axis=0)
    CE2 = Wx2t[9:10]
    W6t = W6.T
    W6a = W6t[0:64]
    W6b = jnp.concatenate([jnp.zeros((3, 128), f32), W6t[64:128]], axis=0)
    W6c = jnp.concatenate([W6t[128:192], jnp.zeros((64, 128), f32)], axis=0)
    W7t = W7.T

    h1_w = [WA, WB, CE, W2t, W3t, W4a, W4b, W5t]
    h2_w = [WA2, WB2, CE2, W6a, W6b, W6c, W7t]

    z128 = jnp.zeros((2, 128), f32)
    z64 = jnp.zeros((2, 64), f32)

    idx1, dsel1 = _knn(warped_xyz, f2_xyz, N)
    g1 = _gather(table1, idx1.reshape(-1)).reshape(B, S * K, 128)
    d1 = dsel1.reshape(B, S * K, 1)

    ab = [z128, z64, z64, z64, z128, z64]
    sh1, sp1 = _stats_out(128)
    shx, spx = _stats_out(64)
    sht, spt = _stats_out(192)
    st = _run_half(_h1_body, 0, wcat, g1, d1, h1_w, ab, sht, spt, TQ)
    ab[0] = _make_ab(st[:, 0:128], g1_, b1_, count)
    ab[1] = _make_ab(st[:, 128:192], gx1, bx1, count)
    st = _run_half(_h1_body, 1, wcat, g1, d1, h1_w, ab, shx, spx, TQ)
    ab[2] = _make_ab(st, g2_, b2_, count)
    st = _run_half(_h1_body, 2, wcat, g1, d1, h1_w, ab, shx, spx, TQ)
    ab[3] = _make_ab(st, g3_, b3_, count)
    st = _run_half(_h1_body, 3, wcat, g1, d1, h1_w, ab, sh1, sp1, TQ)
    ab[4] = _make_ab(st, g4_, b4_, count)
    st = _run_half(_h1_body, 4, wcat, g1, d1, h1_w, ab, shx, spx, TQ)
    ab[5] = _make_ab(st, g5_, b5_, count)
    pf = _run_half(_h1_body, 5, wcat, g1, d1, h1_w, ab,
                   jax.ShapeDtypeStruct((B, S, 128), f32),
                   pl.BlockSpec((1, TQ, 128), lambda b, t: (b, t, 0)), TQ)

    idx2, dsel2 = _knn(warped_xyz, warped_xyz, S)
    g2 = _gather(pf.reshape(B * S, 128), idx2.reshape(-1)).reshape(B, S * K, 128)
    d2 = dsel2.reshape(B, S * K, 1)

    ab2_ = [z64, z128, z64]
    st = _run_half(_h2_body, 0, wcat, g2, d2, h2_w, ab2_, shx, spx, TQ)
    ab2_[0] = _make_ab(st, gx2, bx2, count)
    st = _run_half(_h2_body, 1, wcat, g2, d2, h2_w, ab2_, sh1, sp1, TQ)
    ab2_[1] = _make_ab(st, g6_, b6_, count)
    st = _run_half(_h2_body, 2, wcat, g2, d2, h2_w, ab2_, shx, spx, TQ)
    ab2_[2] = _make_ab(st, g7_, b7_, count)
    out = _run_half(_h2_body, 3, wcat, g2, d2, h2_w, ab2_,
                    jax.ShapeDtypeStruct((B, S, 64), f32),
                    pl.BlockSpec((1, TQ, 64), lambda b, t: (b, t, 0)), TQ)

    return jnp.transpose(out, (0, 2, 1))


# speedup vs baseline: 1.1116x; 1.0374x over previous
import functools

import jax
import jax.numpy as jnp
from jax.experimental import pallas as pl
from jax.experimental.pallas import tpu as pltpu
from jax.experimental.pallas import tpu_sc as plsc

_INTERPRET = False

K = 16
EPS = 1e-5


def _knn_body(nsample, n_db, offset_scale, refs):
    q_ref, db_ref, idx_ref, dsel_ref = refs
    b = pl.program_id(0)
    q = q_ref[0]
    db = db_ref[0]
    qn = jnp.sum(q * q, axis=0)[:, None]
    dbn = jnp.sum(db * db, axis=0)[None, :]
    qd = jax.lax.dot_general(q, db, (((0,), (0,)), ((), ())),
                             preferred_element_type=jnp.float32)
    d = qn + dbn - 2.0 * qd
    iota = jax.lax.broadcasted_iota(jnp.int32, d.shape, 1)
    cols = []
    dcols = []
    for _ in range(nsample):
        m = jnp.min(d, axis=1, keepdims=True)
        cand = jnp.where(d <= m, iota, n_db)
        ik = jnp.min(cand, axis=1)
        cols.append(ik[:, None])
        dcols.append(jnp.maximum(m, 0.0))
        d = jnp.where(iota == ik[:, None], jnp.float32(jnp.inf), d)
    idx = jnp.concatenate(cols, axis=1)
    idx_ref[0] = idx + b * offset_scale
    euc = jnp.sqrt(jnp.concatenate(dcols, axis=1) + 1e-20)
    dsel_ref[0] = euc


def _knn(query_xyz, db_xyz, offset_scale):
    B, _, S = query_xyz.shape
    N = db_xyz.shape[2]
    TS = min(256, S)
    body = functools.partial(_knn_body, K, N, offset_scale)
    return pl.pallas_call(
        lambda *refs: body(refs),
        grid=(B, S // TS),
        in_specs=[
            pl.BlockSpec((1, 3, TS), lambda b, t: (b, 0, t)),
            pl.BlockSpec((1, 3, N), lambda b, t: (b, 0, 0)),
        ],
        out_specs=[pl.BlockSpec((1, TS, K), lambda b, t: (b, t, 0)),
                   pl.BlockSpec((1, TS, K), lambda b, t: (b, t, 0))],
        out_shape=[jax.ShapeDtypeStruct((B, S, K), jnp.int32),
                   jax.ShapeDtypeStruct((B, S, K), jnp.float32)],
        interpret=_INTERPRET,
    )(query_xyz, db_xyz)


def _gather(table, idx_flat):
    M = idx_flat.shape[0]
    D = table.shape[1]
    NC = 2
    NW = 32
    per_w = M // NW
    CH = 128
    mesh = plsc.VectorSubcoreMesh(core_axis_name="c", subcore_axis_name="s")

    @functools.partial(
        pl.kernel, mesh=mesh,
        out_type=jax.ShapeDtypeStruct((M, D), jnp.float32),
        scratch_types=[
            pltpu.VMEM((CH,), jnp.int32),
            pltpu.VMEM((CH, D), jnp.float32),
            pltpu.SemaphoreType.DMA,
        ],
    )
    def k(table_hbm, idx_hbm, out_hbm, idx_v, rows_v, sem):
        wid = jax.lax.axis_index("s") * NC + jax.lax.axis_index("c")
        base = wid * per_w

        def body(i, carry):
            off = base + i * CH
            pltpu.sync_copy(idx_hbm.at[pl.ds(off, CH)], idx_v)
            pltpu.async_copy(table_hbm.at[idx_v], rows_v, sem).wait()
            pltpu.sync_copy(rows_v, out_hbm.at[pl.ds(off, CH)])
            return carry

        jax.lax.fori_loop(0, per_w // CH, body, 0)

    return k(table, idx_flat)


def _act(z, ab_ref):
    a = ab_ref[0:1, :]
    b = ab_ref[1:2, :]
    return jnp.maximum(z * a + b, 0.0)


def _accum_stats(z, out_ref):
    sz = jnp.sum(z, axis=0)
    sq = jnp.sum(z * z, axis=0)
    st = jnp.concatenate([sz[None, :], sq[None, :]], axis=0)
    first = (pl.program_id(0) == 0) & (pl.program_id(1) == 0)

    @pl.when(first)
    def _():
        out_ref[...] = st

    @pl.when(jnp.logical_not(first))
    def _():
        out_ref[...] = out_ref[...] + st


def _dot(x, w):
    return jax.lax.dot_general(x, w, (((1,), (0,)), ((), ())),
                               preferred_element_type=jnp.float32)


def _qadd(zr, zq, TQ):
    C = zr.shape[1]
    return (zr.reshape(TQ, K, C) + zq[:, None, :]).reshape(TQ * K, C)


def _segsum(x, TQ):
    C = x.shape[1]
    return jnp.sum(x.reshape(TQ, K, C), axis=1)


def _t_fused(wcat_ref, g_ref, d_ref, WA, WB, CE, TQ):
    wcat = wcat_ref[0]
    g = g_ref[0]
    euc = d_ref[0]
    return _qadd(_dot(g, WB[...]) + euc * CE[...], _dot(wcat, WA[...]), TQ)


def _h1p0(TQ, refs):
    (wcat_ref, g_ref, d_ref, WA, WB, CE, st_ref) = refs
    t = _t_fused(wcat_ref, g_ref, d_ref, WA, WB, CE, TQ)
    _accum_stats(t, st_ref)


def _h1p1(TQ, refs):
    (wcat_ref, g_ref, d_ref, WA, WB, CE, W2, ab1, abx,
     st_ref, y1_ref, e_ref) = refs
    t = _t_fused(wcat_ref, g_ref, d_ref, WA, WB, CE, TQ)
    y1 = _act(t[:, 0:128], ab1)
    e = _act(t[:, 128:192], abx)
    z2 = _dot(y1, W2[...])
    _accum_stats(z2, st_ref)
    y1_ref[0] = y1
    e_ref[0] = e


def _h1p2(TQ, refs):
    (y1_ref, W2, W3, ab2, st_ref, z3_ref) = refs
    z2 = _dot(y1_ref[0], W2[...])
    y2 = _act(z2, ab2)
    z3 = _dot(y2, W3[...])
    _accum_stats(z3, st_ref)
    z3_ref[0] = z3


def _h1p3(TQ, refs):
    (z3_ref, e_ref, W4a, W4b, ab3, st_ref) = refs
    y3 = _act(z3_ref[0], ab3)
    z4 = _dot(e_ref[0], W4a[...]) + _dot(y3, W4b[...])
    _accum_stats(z4, st_ref)


def _h1p4(TQ, refs):
    (z3_ref, e_ref, W4a, W4b, W5, ab3, ab4, st_ref, z5_ref) = refs
    y3 = _act(z3_ref[0], ab3)
    z4 = _dot(e_ref[0], W4a[...]) + _dot(y3, W4b[...])
    y4 = _act(z4, ab4)
    z5 = _dot(y4, W5[...])
    _accum_stats(z5, st_ref)
    z5_ref[0] = z5


def _h1p5(TQ, refs):
    (wcat_ref, z3_ref, z5_ref, ab3, ab5, out_ref) = refs
    y3 = _act(z3_ref[0], ab3)
    y5 = _act(z5_ref[0], ab5)
    w = jnp.exp(y5)
    denom = _segsum(w, TQ)
    num = _segsum(w * y3, TQ)
    outq = num / denom
    pad = jnp.zeros((TQ, 61), jnp.float32)
    out_ref[0] = jnp.concatenate([outq, wcat_ref[0][:, 0:3], pad], axis=1)


def _h2p0(TQ, refs):
    (wcat_ref, g_ref, d_ref, WA2, WB2, CE2, st_ref) = refs
    ex = _t_fused(wcat_ref, g_ref, d_ref, WA2, WB2, CE2, TQ)
    _accum_stats(ex, st_ref)


def _h2p1(TQ, refs):
    (wcat_ref, g_ref, d_ref, WA2, WB2, CE2, W6a, W6b, W6c, abx,
     st_ref, z6_ref) = refs
    ex = _t_fused(wcat_ref, g_ref, d_ref, WA2, WB2, CE2, TQ)
    e = _act(ex, abx)
    z6 = _qadd(_dot(e, W6a[...]) + _dot(g_ref[0], W6c[...]),
               _dot(wcat_ref[0], W6b[...]), TQ)
    _accum_stats(z6, st_ref)
    z6_ref[0] = z6


def _h2p2(TQ, refs):
    (z6_ref, W7, ab6, st_ref, z7_ref) = refs
    y6 = _act(z6_ref[0], ab6)
    z7 = _dot(y6, W7[...])
    _accum_stats(z7, st_ref)
    z7_ref[0] = z7


def _h2p3(TQ, refs):
    (g_ref, z7_ref, ab7, out_ref) = refs
    y7 = _act(z7_ref[0], ab7)
    w = jnp.exp(y7)
    gf = g_ref[0][:, 0:64]
    denom = _segsum(w, TQ)
    num = _segsum(w * gf, TQ)
    out_ref[0] = num / denom


def _full_spec(shape):
    nd = len(shape)
    return pl.BlockSpec(shape, lambda b, t, _n=nd: (0,) * _n)


def _pcall(body_fn, TQ, B, S, ins, out_shapes, out_specs):
    in_specs = []
    args = []
    for a, kind in ins:
        if kind == "q":
            C = a.shape[2]
            in_specs.append(pl.BlockSpec((1, TQ, C), lambda b, t: (b, t, 0)))
        elif kind == "r":
            C = a.shape[2]
            in_specs.append(pl.BlockSpec((1, TQ * K, C), lambda b, t: (b, t, 0)))
        else:
            in_specs.append(_full_spec(a.shape))
        args.append(a)
    return pl.pallas_call(
        lambda *refs: body_fn(TQ, refs),
        grid=(B, S // TQ),
        in_specs=in_specs,
        out_specs=out_specs,
        out_shape=out_shapes,
        interpret=_INTERPRET,
    )(*args)


def _make_ab(stats, gamma, beta, count):
    s, q = stats[0], stats[1]
    mean = s / count
    var = q / count - mean * mean
    a = gamma / jnp.sqrt(var + EPS)
    b = beta - mean * a
    return jnp.stack([a, b])


def kernel(warped_xyz, warped_points, f2_xyz, f2_points,
           mlp1_params, xyz1_params, xyz2_params, mlp2_params, mlp3_params):
    B, _, S = warped_xyz.shape
    N = f2_xyz.shape[2]
    f32 = jnp.float32
    TQ = min(128, S)
    count = float(B * S * K)

    wxyz_t = jnp.transpose(warped_xyz, (0, 2, 1))
    wcat = jnp.concatenate([wxyz_t, jnp.transpose(warped_points, (0, 2, 1))],
                           axis=2)
    table1 = jnp.concatenate(
        [jnp.transpose(f2_points, (0, 2, 1)),
         jnp.transpose(f2_xyz, (0, 2, 1)),
         jnp.zeros((B, N, 61), f32)], axis=2).reshape(B * N, 128)

    (W1, g1_, b1_), (W2, g2_, b2_), (W3, g3_, b3_) = mlp1_params
    ((Wx1, gx1, bx1),) = xyz1_params
    ((Wx2, gx2, bx2),) = xyz2_params
    (W4, g4_, b4_), (W5, g5_, b5_) = mlp2_params
    (W6, g6_, b6_), (W7, g7_, b7_) = mlp3_params
    W1t = W1.T
    Wx1t = Wx1.T
    WA = jnp.concatenate([
        jnp.concatenate([W1t[0:3] - W1t[6:9], W1t[10:74]], axis=0),
        jnp.concatenate([Wx1t[0:3] - Wx1t[6:9], jnp.zeros((64, 64), f32)],
                        axis=0)], axis=1)
    WB = jnp.concatenate([
        jnp.concatenate([W1t[74:138], W1t[3:6] + W1t[6:9],
                         jnp.zeros((61, 128), f32)], axis=0),
        jnp.concatenate([jnp.zeros((64, 64), f32), Wx1t[3:6] + Wx1t[6:9],
                         jnp.zeros((61, 64), f32)], axis=0)], axis=1)
    CE = jnp.concatenate([W1t[9:10], Wx1t[9:10]], axis=1)
    W2t, W3t = W2.T, W3.T
    W4t = W4.T
    W4a, W4b = W4t[0:64], W4t[64:128]
    W5t = W5.T
    Wx2t = Wx2.T
    WA2 = jnp.concatenate([Wx2t[0:3] - Wx2t[6:9],
                           jnp.zeros((64, 64), f32)], axis=0)
    WB2 = jnp.concatenate([jnp.zeros((64, 64), f32), Wx2t[3:6] + Wx2t[6:9],
                           jnp.zeros((61, 64), f32)], axis=0)
    CE2 = Wx2t[9:10]
    W6t = W6.T
    W6a = W6t[0:64]
    W6b = jnp.concatenate([jnp.zeros((3, 128), f32), W6t[64:128]], axis=0)
    W6c = jnp.concatenate([W6t[128:192], jnp.zeros((64, 128), f32)], axis=0)
    W7t = W7.T

    def ss(C):
        return (jax.ShapeDtypeStruct((2, C), f32),
                pl.BlockSpec((2, C), lambda b, t: (0, 0)))

    def rs(C):
        return (jax.ShapeDtypeStruct((B, S * K, C), f32),
                pl.BlockSpec((1, TQ * K, C), lambda b, t: (b, t, 0)))

    def qs(C):
        return (jax.ShapeDtypeStruct((B, S, C), f32),
                pl.BlockSpec((1, TQ, C), lambda b, t: (b, t, 0)))

    def pc(body, ins, *outs):
        return _pcall(body, TQ, B, S, ins,
                      [o[0] for o in outs] if len(outs) > 1 else outs[0][0],
                      [o[1] for o in outs] if len(outs) > 1 else outs[0][1])

    idx1, euc1 = _knn(warped_xyz, f2_xyz, N)
    g1 = _gather(table1, idx1.reshape(-1)).reshape(B, S * K, 128)
    d1 = euc1.reshape(B, S * K, 1)
    wq1 = [(wcat, "q"), (g1, "r"), (d1, "r")]
    fused1 = [(WA, "w"), (WB, "w"), (CE, "w")]

    st = pc(_h1p0, wq1 + fused1, ss(192))
    ab1 = _make_ab(st[:, 0:128], g1_, b1_, count)
    abx = _make_ab(st[:, 128:192], gx1, bx1, count)
    st, y1c, ec = pc(_h1p1, wq1 + fused1 + [(W2t, "w"), (ab1, "w"), (abx, "w")],
                     ss(64), rs(128), rs(64))
    ab2 = _make_ab(st, g2_, b2_, count)
    st, z3c = pc(_h1p2, [(y1c, "r"), (W2t, "w"), (W3t, "w"), (ab2, "w")],
                 ss(64), rs(64))
    ab3 = _make_ab(st, g3_, b3_, count)
    st = pc(_h1p3, [(z3c, "r"), (ec, "r"), (W4a, "w"), (W4b, "w"), (ab3, "w")],
            ss(128))
    ab4 = _make_ab(st, g4_, b4_, count)
    st, z5c = pc(_h1p4, [(z3c, "r"), (ec, "r"), (W4a, "w"), (W4b, "w"),
                         (W5t, "w"), (ab3, "w"), (ab4, "w")], ss(64), rs(64))
    ab5 = _make_ab(st, g5_, b5_, count)
    pf = pc(_h1p5, [(wcat, "q"), (z3c, "r"), (z5c, "r"), (ab3, "w"),
                    (ab5, "w")], qs(128))

    idx2, euc2 = _knn(warped_xyz, warped_xyz, S)
    g2 = _gather(pf.reshape(B * S, 128), idx2.reshape(-1)).reshape(B, S * K, 128)
    d2 = euc2.reshape(B, S * K, 1)
    wq2 = [(wcat, "q"), (g2, "r"), (d2, "r")]
    fused2 = [(WA2, "w"), (WB2, "w"), (CE2, "w")]

    st = pc(_h2p0, wq2 + fused2, ss(64))
    abx2 = _make_ab(st, gx2, bx2, count)
    st, z6c = pc(_h2p1, wq2 + fused2 + [(W6a, "w"), (W6b, "w"), (W6c, "w"),
                                        (abx2, "w")], ss(128), rs(128))
    ab6 = _make_ab(st, g6_, b6_, count)
    st, z7c = pc(_h2p2, [(z6c, "r"), (W7t, "w"), (ab6, "w")], ss(64), rs(64))
    ab7 = _make_ab(st, g7_, b7_, count)
    out = pc(_h2p3, [(g2, "r"), (z7c, "r"), (ab7, "w")], qs(64))

    return jnp.transpose(out, (0, 2, 1))
